# Initial kernel scaffold; baseline (speedup 1.0000x reference)
import jax
import jax.numpy as jnp
from jax.experimental import pallas as pl


def kernel(k, E_blocks, centroids, decoded_centroids):
    raise NotImplementedError("write your pallas kernel here")


# speedup vs baseline: 6.5146x; 6.5146x over previous
import functools

import jax
import jax.numpy as jnp
from jax import lax
from jax.experimental import pallas as pl
from jax.experimental.pallas import tpu as pltpu

HEAD_DIM = 128
BLOCK_DIM = 8
N_BLOCKS = HEAD_DIM // BLOCK_DIM
N_CENT = 256
WIDE = N_BLOCKS * N_CENT

TILE = 256


def _block_diag(mats):
    G, a, b = mats.shape
    eye = jnp.eye(G, dtype=mats.dtype)
    return jnp.einsum('gab,gh->gahb', mats, eye).reshape(G * a, G * b)


def _vq_kernel(k_ref, e_ref, c_ref, c2_ref, ones_ref, dc_ref,
               idx_ref, khat_ref):
    x = k_ref[...]
    kshaped = jnp.dot(x, e_ref[...], preferred_element_type=jnp.float32,
                      precision=lax.Precision.DEFAULT)
    cross = jnp.dot(kshaped, c_ref[...], preferred_element_type=jnp.float32,
                    precision=lax.Precision.DEFAULT)
    ks2 = jnp.dot(kshaped * kshaped, ones_ref[...],
                  preferred_element_type=jnp.float32,
                  precision=lax.Precision.HIGHEST)
    onehots = []
    for b in range(N_BLOCKS):
        sl = slice(b * N_CENT, (b + 1) * N_CENT)
        d2 = ks2[:, b:b + 1] + c2_ref[:, sl] - 2.0 * cross[:, sl]
        ib = jnp.argmin(d2, axis=1).astype(jnp.int32)
        idx_ref[:, b:b + 1] = ib[:, None]
        onehots.append(
            (lax.broadcasted_iota(jnp.int32, (TILE, N_CENT), 1)
             == ib[:, None]).astype(jnp.float32))
    onehot = jnp.concatenate(onehots, axis=1)
    khat_ref[...] = jnp.dot(onehot, dc_ref[...],
                            preferred_element_type=jnp.float32,
                            precision=lax.Precision.HIGHEST)


@jax.jit
def kernel(k, E_blocks, centroids, decoded_centroids):
    batch_shape = k.shape[:-1]
    kf = k.reshape(-1, HEAD_DIM).astype(jnp.float32)
    n = kf.shape[0]

    e_bd = _block_diag(jnp.transpose(E_blocks, (0, 2, 1)))
    c_bd = _block_diag(jnp.transpose(centroids, (0, 2, 1)))
    dc_bd = _block_diag(decoded_centroids)
    c2 = jnp.sum(centroids * centroids, axis=-1).reshape(1, WIDE)
    ones_bd = _block_diag(jnp.ones((N_BLOCKS, BLOCK_DIM, 1), jnp.float32))

    grid = (n // TILE,)
    const = lambda i: (0, 0)
    idx, khat = pl.pallas_call(
        _vq_kernel,
        grid=grid,
        in_specs=[
            pl.BlockSpec((TILE, HEAD_DIM), lambda i: (i, 0)),
            pl.BlockSpec((HEAD_DIM, HEAD_DIM), const),
            pl.BlockSpec((HEAD_DIM, WIDE), const),
            pl.BlockSpec((1, WIDE), const),
            pl.BlockSpec((HEAD_DIM, N_BLOCKS), const),
            pl.BlockSpec((WIDE, HEAD_DIM), const),
        ],
        out_specs=[
            pl.BlockSpec((TILE, N_BLOCKS), lambda i: (i, 0)),
            pl.BlockSpec((TILE, HEAD_DIM), lambda i: (i, 0)),
        ],
        out_shape=[
            jax.ShapeDtypeStruct((n, N_BLOCKS), jnp.int32),
            jax.ShapeDtypeStruct((n, HEAD_DIM), jnp.float32),
        ],
        compiler_params=pltpu.CompilerParams(
            dimension_semantics=("arbitrary",),
        ),
    )(kf, e_bd, c_bd, c2, ones_bd, dc_bd)

    return (idx.reshape(*batch_shape, N_BLOCKS),
            khat.reshape(*batch_shape, HEAD_DIM))


# speedup vs baseline: 8.3792x; 1.2862x over previous
import functools

import jax
import jax.numpy as jnp
from jax import lax
from jax.experimental import pallas as pl
from jax.experimental.pallas import tpu as pltpu

HEAD_DIM = 128
BLOCK_DIM = 8
N_BLOCKS = HEAD_DIM // BLOCK_DIM
N_CENT = 256
WIDE = N_BLOCKS * N_CENT

TILE = 256


def _block_diag(mats):
    G, a, b = mats.shape
    eye = jnp.eye(G, dtype=mats.dtype)
    return jnp.einsum('gab,gh->gahb', mats, eye).reshape(G * a, G * b)


def _vq_kernel(k_ref, e_ref, c_ref, c2_ref, ones_ref, dc_ref,
               idx_ref, khat_ref):
    x = k_ref[...]
    kshaped = jnp.dot(x, e_ref[...], preferred_element_type=jnp.float32,
                      precision=lax.Precision.DEFAULT)
    cross = jnp.dot(kshaped, c_ref[...], preferred_element_type=jnp.float32,
                    precision=lax.Precision.DEFAULT)
    ks2 = jnp.dot(kshaped * kshaped, ones_ref[...],
                  preferred_element_type=jnp.float32,
                  precision=lax.Precision.HIGHEST)
    cols = []
    onehots = []
    for b in range(N_BLOCKS):
        sl = slice(b * N_CENT, (b + 1) * N_CENT)
        d2 = ks2[:, b:b + 1] + c2_ref[:, sl] - 2.0 * cross[:, sl]
        ib = jnp.argmin(d2, axis=1).astype(jnp.int32)
        cols.append(ib[:, None])
        onehots.append(
            (lax.broadcasted_iota(jnp.int32, (TILE, N_CENT), 1)
             == ib[:, None]).astype(jnp.float32))
    idx_ref[...] = jnp.concatenate(cols, axis=1)
    onehot = jnp.concatenate(onehots, axis=1)
    khat_ref[...] = jnp.dot(onehot, dc_ref[...],
                            preferred_element_type=jnp.float32,
                            precision=lax.Precision.DEFAULT)


@jax.jit
def kernel(k, E_blocks, centroids, decoded_centroids):
    batch_shape = k.shape[:-1]
    kf = k.reshape(-1, HEAD_DIM).astype(jnp.float32)
    n = kf.shape[0]

    e_bd = _block_diag(jnp.transpose(E_blocks, (0, 2, 1)))
    c_bd = _block_diag(jnp.transpose(centroids, (0, 2, 1)))
    dc_bd = _block_diag(decoded_centroids)
    c2 = jnp.sum(centroids * centroids, axis=-1).reshape(1, WIDE)
    ones_bd = _block_diag(jnp.ones((N_BLOCKS, BLOCK_DIM, 1), jnp.float32))

    grid = (n // TILE,)
    const = lambda i: (0, 0)
    idx, khat = pl.pallas_call(
        _vq_kernel,
        grid=grid,
        in_specs=[
            pl.BlockSpec((TILE, HEAD_DIM), lambda i: (i, 0)),
            pl.BlockSpec((HEAD_DIM, HEAD_DIM), const),
            pl.BlockSpec((HEAD_DIM, WIDE), const),
            pl.BlockSpec((1, WIDE), const),
            pl.BlockSpec((HEAD_DIM, N_BLOCKS), const),
            pl.BlockSpec((WIDE, HEAD_DIM), const),
        ],
        out_specs=[
            pl.BlockSpec((TILE, N_BLOCKS), lambda i: (i, 0)),
            pl.BlockSpec((TILE, HEAD_DIM), lambda i: (i, 0)),
        ],
        out_shape=[
            jax.ShapeDtypeStruct((n, N_BLOCKS), jnp.int32),
            jax.ShapeDtypeStruct((n, HEAD_DIM), jnp.float32),
        ],
        compiler_params=pltpu.CompilerParams(
            dimension_semantics=("arbitrary",),
        ),
    )(kf, e_bd, c_bd, c2, ones_bd, dc_bd)

    return (idx.reshape(*batch_shape, N_BLOCKS),
            khat.reshape(*batch_shape, HEAD_DIM))


# speedup vs baseline: 9.1312x; 1.0898x over previous
import functools

import jax
import jax.numpy as jnp
from jax import lax
from jax.experimental import pallas as pl
from jax.experimental.pallas import tpu as pltpu

HEAD_DIM = 128
BLOCK_DIM = 8
N_BLOCKS = HEAD_DIM // BLOCK_DIM
N_CENT = 256
WIDE = N_BLOCKS * N_CENT

TILE = 512


def _block_diag(mats):
    G, a, b = mats.shape
    eye = jnp.eye(G, dtype=mats.dtype)
    return jnp.einsum('gab,gh->gahb', mats, eye).reshape(G * a, G * b)


def _vq_kernel(k_ref, e_ref, c_ref, c2_ref, ones_ref, dc_ref,
               idx_ref, khat_ref):
    x = k_ref[...]
    kshaped = jnp.dot(x, e_ref[...], preferred_element_type=jnp.float32,
                      precision=lax.Precision.DEFAULT)
    cross = jnp.dot(kshaped, c_ref[...], preferred_element_type=jnp.float32,
                    precision=lax.Precision.DEFAULT)
    ks2 = jnp.dot(kshaped * kshaped, ones_ref[...],
                  preferred_element_type=jnp.float32,
                  precision=lax.Precision.HIGHEST)
    cols = []
    onehots = []
    for b in range(N_BLOCKS):
        sl = slice(b * N_CENT, (b + 1) * N_CENT)
        d2 = ks2[:, b:b + 1] + c2_ref[:, sl] - 2.0 * cross[:, sl]
        ib = jnp.argmin(d2, axis=1).astype(jnp.int32)
        cols.append(ib[:, None])
        onehots.append(
            (lax.broadcasted_iota(jnp.int32, (TILE, N_CENT), 1)
             == ib[:, None]).astype(jnp.float32))
    idx_ref[...] = jnp.concatenate(cols, axis=1)
    onehot = jnp.concatenate(onehots, axis=1)
    khat_ref[...] = jnp.dot(onehot, dc_ref[...],
                            preferred_element_type=jnp.float32,
                            precision=lax.Precision.DEFAULT)


@jax.jit
def kernel(k, E_blocks, centroids, decoded_centroids):
    batch_shape = k.shape[:-1]
    kf = k.reshape(-1, HEAD_DIM).astype(jnp.float32)
    n = kf.shape[0]

    e_bd = _block_diag(jnp.transpose(E_blocks, (0, 2, 1)))
    c_bd = _block_diag(jnp.transpose(centroids, (0, 2, 1)))
    dc_bd = _block_diag(decoded_centroids)
    c2 = jnp.sum(centroids * centroids, axis=-1).reshape(1, WIDE)
    ones_bd = _block_diag(jnp.ones((N_BLOCKS, BLOCK_DIM, 1), jnp.float32))

    grid = (n // TILE,)
    const = lambda i: (0, 0)
    idx, khat = pl.pallas_call(
        _vq_kernel,
        grid=grid,
        in_specs=[
            pl.BlockSpec((TILE, HEAD_DIM), lambda i: (i, 0)),
            pl.BlockSpec((HEAD_DIM, HEAD_DIM), const),
            pl.BlockSpec((HEAD_DIM, WIDE), const),
            pl.BlockSpec((1, WIDE), const),
            pl.BlockSpec((HEAD_DIM, N_BLOCKS), const),
            pl.BlockSpec((WIDE, HEAD_DIM), const),
        ],
        out_specs=[
            pl.BlockSpec((TILE, N_BLOCKS), lambda i: (i, 0)),
            pl.BlockSpec((TILE, HEAD_DIM), lambda i: (i, 0)),
        ],
        out_shape=[
            jax.ShapeDtypeStruct((n, N_BLOCKS), jnp.int32),
            jax.ShapeDtypeStruct((n, HEAD_DIM), jnp.float32),
        ],
        compiler_params=pltpu.CompilerParams(
            dimension_semantics=("arbitrary",),
        ),
    )(kf, e_bd, c_bd, c2, ones_bd, dc_bd)

    return (idx.reshape(*batch_shape, N_BLOCKS),
            khat.reshape(*batch_shape, HEAD_DIM))


# speedup vs baseline: 11.0723x; 1.2126x over previous
import functools

import jax
import jax.numpy as jnp
from jax import lax
from jax.experimental import pallas as pl
from jax.experimental.pallas import tpu as pltpu

HEAD_DIM = 128
BLOCK_DIM = 8
N_BLOCKS = HEAD_DIM // BLOCK_DIM
N_CENT = 256
WIDE = N_BLOCKS * N_CENT

TILE = 1024


def _block_diag(mats):
    G, a, b = mats.shape
    eye = jnp.eye(G, dtype=mats.dtype)
    return jnp.einsum('gab,gh->gahb', mats, eye).reshape(G * a, G * b)


def _vq_kernel(k_ref, e_ref, c_ref, c2_ref, ones_ref, dc_ref,
               idx_ref, khat_ref):
    x = k_ref[...]
    kshaped = jnp.dot(x, e_ref[...], preferred_element_type=jnp.float32,
                      precision=lax.Precision.DEFAULT)
    cross = jnp.dot(kshaped, c_ref[...], preferred_element_type=jnp.float32,
                    precision=lax.Precision.DEFAULT)
    ks2 = jnp.dot(kshaped * kshaped, ones_ref[...],
                  preferred_element_type=jnp.float32,
                  precision=lax.Precision.HIGHEST)
    cols = []
    onehots = []
    for b in range(N_BLOCKS):
        sl = slice(b * N_CENT, (b + 1) * N_CENT)
        d2 = ks2[:, b:b + 1] + c2_ref[:, sl] - 2.0 * cross[:, sl]
        ib = jnp.argmin(d2, axis=1).astype(jnp.int32)
        cols.append(ib[:, None])
        onehots.append(
            (lax.broadcasted_iota(jnp.int32, (TILE, N_CENT), 1)
             == ib[:, None]).astype(jnp.float32))
    idx_ref[...] = jnp.concatenate(cols, axis=1)
    onehot = jnp.concatenate(onehots, axis=1)
    khat_ref[...] = jnp.dot(onehot, dc_ref[...],
                            preferred_element_type=jnp.float32,
                            precision=lax.Precision.DEFAULT)


@jax.jit
def kernel(k, E_blocks, centroids, decoded_centroids):
    batch_shape = k.shape[:-1]
    kf = k.reshape(-1, HEAD_DIM).astype(jnp.float32)
    n = kf.shape[0]

    e_bd = _block_diag(jnp.transpose(E_blocks, (0, 2, 1)))
    c_bd = _block_diag(jnp.transpose(centroids, (0, 2, 1)))
    dc_bd = _block_diag(decoded_centroids)
    c2 = jnp.sum(centroids * centroids, axis=-1).reshape(1, WIDE)
    ones_bd = _block_diag(jnp.ones((N_BLOCKS, BLOCK_DIM, 1), jnp.float32))

    grid = (n // TILE,)
    const = lambda i: (0, 0)
    idx, khat = pl.pallas_call(
        _vq_kernel,
        grid=grid,
        in_specs=[
            pl.BlockSpec((TILE, HEAD_DIM), lambda i: (i, 0)),
            pl.BlockSpec((HEAD_DIM, HEAD_DIM), const),
            pl.BlockSpec((HEAD_DIM, WIDE), const),
            pl.BlockSpec((1, WIDE), const),
            pl.BlockSpec((HEAD_DIM, N_BLOCKS), const),
            pl.BlockSpec((WIDE, HEAD_DIM), const),
        ],
        out_specs=[
            pl.BlockSpec((TILE, N_BLOCKS), lambda i: (i, 0)),
            pl.BlockSpec((TILE, HEAD_DIM), lambda i: (i, 0)),
        ],
        out_shape=[
            jax.ShapeDtypeStruct((n, N_BLOCKS), jnp.int32),
            jax.ShapeDtypeStruct((n, HEAD_DIM), jnp.float32),
        ],
        compiler_params=pltpu.CompilerParams(
            dimension_semantics=("arbitrary",),
        ),
    )(kf, e_bd, c_bd, c2, ones_bd, dc_bd)

    return (idx.reshape(*batch_shape, N_BLOCKS),
            khat.reshape(*batch_shape, HEAD_DIM))


# speedup vs baseline: 11.9089x; 1.0756x over previous
import functools

import jax
import jax.numpy as jnp
from jax import lax
from jax.experimental import pallas as pl
from jax.experimental.pallas import tpu as pltpu

HEAD_DIM = 128
BLOCK_DIM = 8
N_BLOCKS = HEAD_DIM // BLOCK_DIM
N_CENT = 256
WIDE = N_BLOCKS * N_CENT

TILE = 2048


def _block_diag(mats):
    G, a, b = mats.shape
    eye = jnp.eye(G, dtype=mats.dtype)
    return jnp.einsum('gab,gh->gahb', mats, eye).reshape(G * a, G * b)


def _vq_kernel(k_ref, e_ref, c_ref, c2_ref, ones_ref, dc_ref,
               idx_ref, khat_ref):
    x = k_ref[...]
    kshaped = jnp.dot(x, e_ref[...], preferred_element_type=jnp.float32,
                      precision=lax.Precision.DEFAULT)
    cross = jnp.dot(kshaped, c_ref[...], preferred_element_type=jnp.float32,
                    precision=lax.Precision.DEFAULT)
    ks2 = jnp.dot(kshaped * kshaped, ones_ref[...],
                  preferred_element_type=jnp.float32,
                  precision=lax.Precision.HIGHEST)
    cols = []
    onehots = []
    for b in range(N_BLOCKS):
        sl = slice(b * N_CENT, (b + 1) * N_CENT)
        d2 = ks2[:, b:b + 1] + c2_ref[:, sl] - 2.0 * cross[:, sl]
        ib = jnp.argmin(d2, axis=1).astype(jnp.int32)
        cols.append(ib[:, None])
        onehots.append(
            (lax.broadcasted_iota(jnp.int32, (TILE, N_CENT), 1)
             == ib[:, None]).astype(jnp.float32))
    idx_ref[...] = jnp.concatenate(cols, axis=1)
    onehot = jnp.concatenate(onehots, axis=1)
    khat_ref[...] = jnp.dot(onehot, dc_ref[...],
                            preferred_element_type=jnp.float32,
                            precision=lax.Precision.DEFAULT)


@jax.jit
def kernel(k, E_blocks, centroids, decoded_centroids):
    batch_shape = k.shape[:-1]
    kf = k.reshape(-1, HEAD_DIM).astype(jnp.float32)
    n = kf.shape[0]

    e_bd = _block_diag(jnp.transpose(E_blocks, (0, 2, 1)))
    c_bd = _block_diag(jnp.transpose(centroids, (0, 2, 1)))
    dc_bd = _block_diag(decoded_centroids)
    c2 = jnp.sum(centroids * centroids, axis=-1).reshape(1, WIDE)
    ones_bd = _block_diag(jnp.ones((N_BLOCKS, BLOCK_DIM, 1), jnp.float32))

    grid = (n // TILE,)
    const = lambda i: (0, 0)
    idx, khat = pl.pallas_call(
        _vq_kernel,
        grid=grid,
        in_specs=[
            pl.BlockSpec((TILE, HEAD_DIM), lambda i: (i, 0)),
            pl.BlockSpec((HEAD_DIM, HEAD_DIM), const),
            pl.BlockSpec((HEAD_DIM, WIDE), const),
            pl.BlockSpec((1, WIDE), const),
            pl.BlockSpec((HEAD_DIM, N_BLOCKS), const),
            pl.BlockSpec((WIDE, HEAD_DIM), const),
        ],
        out_specs=[
            pl.BlockSpec((TILE, N_BLOCKS), lambda i: (i, 0)),
            pl.BlockSpec((TILE, HEAD_DIM), lambda i: (i, 0)),
        ],
        out_shape=[
            jax.ShapeDtypeStruct((n, N_BLOCKS), jnp.int32),
            jax.ShapeDtypeStruct((n, HEAD_DIM), jnp.float32),
        ],
        compiler_params=pltpu.CompilerParams(
            dimension_semantics=("arbitrary",),
        ),
    )(kf, e_bd, c_bd, c2, ones_bd, dc_bd)

    return (idx.reshape(*batch_shape, N_BLOCKS),
            khat.reshape(*batch_shape, HEAD_DIM))


# speedup vs baseline: 21.9319x; 1.8416x over previous
import functools

import jax
import jax.numpy as jnp
from jax import lax
from jax.experimental import pallas as pl
from jax.experimental.pallas import tpu as pltpu

HEAD_DIM = 128
BLOCK_DIM = 8
N_BLOCKS = HEAD_DIM // BLOCK_DIM
N_CENT = 256
WIDE = N_BLOCKS * N_CENT

TILE = 2048


def _block_diag(mats):
    G, a, b = mats.shape
    eye = jnp.eye(G, dtype=mats.dtype)
    return jnp.einsum('gab,gh->gahb', mats, eye).reshape(G * a, G * b)


def _vq_kernel(k_ref, e_ref, c_ref, c2h_ref, dc_ref,
               idx_ref, khat_ref):
    x = k_ref[...]
    kshaped = jnp.dot(x, e_ref[...], preferred_element_type=jnp.float32,
                      precision=lax.Precision.DEFAULT)
    cross = jnp.dot(kshaped, c_ref[...], preferred_element_type=jnp.float32,
                    precision=lax.Precision.DEFAULT)
    cols = []
    onehots = []
    for b in range(N_BLOCKS):
        sl = slice(b * N_CENT, (b + 1) * N_CENT)
        s = cross[:, sl] - c2h_ref[:, sl]
        ib = jnp.argmax(s, axis=1).astype(jnp.int32)
        cols.append(ib[:, None])
        onehots.append(
            (lax.broadcasted_iota(jnp.int32, (TILE, N_CENT), 1)
             == ib[:, None]).astype(jnp.float32))
    idx_ref[...] = jnp.concatenate(cols, axis=1)
    onehot = jnp.concatenate(onehots, axis=1)
    khat_ref[...] = jnp.dot(onehot, dc_ref[...],
                            preferred_element_type=jnp.float32,
                            precision=lax.Precision.DEFAULT)


@jax.jit
def kernel(k, E_blocks, centroids, decoded_centroids):
    batch_shape = k.shape[:-1]
    kf = k.reshape(-1, HEAD_DIM).astype(jnp.float32)
    n = kf.shape[0]

    e_bd = _block_diag(jnp.transpose(E_blocks, (0, 2, 1)))
    c_bd = _block_diag(jnp.transpose(centroids, (0, 2, 1)))
    dc_bd = _block_diag(decoded_centroids)
    c2h = 0.5 * jnp.sum(centroids * centroids, axis=-1).reshape(1, WIDE)

    grid = (n // TILE,)
    const = lambda i: (0, 0)
    idx, khat = pl.pallas_call(
        _vq_kernel,
        grid=grid,
        in_specs=[
            pl.BlockSpec((TILE, HEAD_DIM), lambda i: (i, 0)),
            pl.BlockSpec((HEAD_DIM, HEAD_DIM), const),
            pl.BlockSpec((HEAD_DIM, WIDE), const),
            pl.BlockSpec((1, WIDE), const),
            pl.BlockSpec((WIDE, HEAD_DIM), const),
        ],
        out_specs=[
            pl.BlockSpec((TILE, N_BLOCKS), lambda i: (i, 0)),
            pl.BlockSpec((TILE, HEAD_DIM), lambda i: (i, 0)),
        ],
        out_shape=[
            jax.ShapeDtypeStruct((n, N_BLOCKS), jnp.int32),
            jax.ShapeDtypeStruct((n, HEAD_DIM), jnp.float32),
        ],
        compiler_params=pltpu.CompilerParams(
            dimension_semantics=("arbitrary",),
        ),
    )(kf, e_bd, c_bd, c2h, dc_bd)

    return (idx.reshape(*batch_shape, N_BLOCKS),
            khat.reshape(*batch_shape, HEAD_DIM))
